# Initial kernel scaffold; baseline (speedup 1.0000x reference)
import jax
import jax.numpy as jnp
from jax.experimental import pallas as pl


def kernel(a_feature, b_feature, edge_index):
    raise NotImplementedError("write your pallas kernel here")


# speedup vs baseline: 12.5276x; 12.5276x over previous
import functools

import jax
import jax.numpy as jnp
from jax import lax
from jax.experimental import pallas as pl
from jax.experimental.pallas import tpu as pltpu
from jax.experimental.pallas import tpu_sc as plsc

N = 50000
D = 64
E = 800000

NP = 50176
SP = NP // 16
HALF = NP // 2
SD = HALF // 16
RB = 112
RBS = 32

EPAD = 819200
EPT = EPAD // 16
G = 256
NG = EPT // G

PADV = NP - 1
TRASH = HALF
ACC_ROWS = HALF + 1

_mesh = plsc.VectorSubcoreMesh(core_axis_name="c", subcore_axis_name="s")


@functools.partial(
    pl.kernel,
    out_type=(
        jax.ShapeDtypeStruct((2 * NP, D), jnp.float32),
        jax.ShapeDtypeStruct((2 * NP,), jnp.float32),
    ),
    mesh=_mesh,
    scratch_types=[
        pltpu.VMEM_SHARED((NP,), jnp.float32),
        pltpu.VMEM((G,), jnp.int32),
        pltpu.VMEM((G,), jnp.float32),
        pltpu.VMEM((SP,), jnp.float32),
        pltpu.VMEM((SP,), jnp.float32),
        pltpu.VMEM((RB, D), jnp.float32),
        pltpu.VMEM((RB, D), jnp.float32),
        pltpu.SemaphoreType.DMA,
    ],
)
def _prep(keys_hbm, feats_hbm, w_hbm, inv_hbm,
          deg, kv, ones, dvm, ivm, fvm, wvm, sem):
    c = lax.axis_index("c")
    s = lax.axis_index("s")
    zero16 = jnp.zeros((16,), jnp.float32)

    def z_body(j, _):
        dvm[pl.ds(j * 16, 16)] = zero16
        return _
    lax.fori_loop(0, SP // 16, z_body, None)

    def o_body(j, _):
        ones[pl.ds(j * 16, 16)] = zero16 + 1.0
        return _
    lax.fori_loop(0, G // 16, o_body, None)

    pltpu.sync_copy(dvm, deg.at[pl.ds(s * SP, SP)])
    plsc.subcore_barrier()

    kbase = c * EPAD + s * EPT

    def h_body(g, _):
        pltpu.sync_copy(keys_hbm.at[pl.ds(kbase + g * G, G)], kv)
        pltpu.sync_copy(ones, deg.at[kv], add=True)
        return _
    lax.fori_loop(0, NG, h_body, None)
    plsc.subcore_barrier()

    pltpu.sync_copy(deg.at[pl.ds(s * SP, SP)], dvm)

    def n_body(j, _):
        sl = pl.ds(j * 16, 16)
        d = dvm[sl]
        di = lax.bitcast_convert_type(d, jnp.int32)
        y = lax.bitcast_convert_type(
            0x5F3759DF - lax.shift_right_logical(di, 1), jnp.float32)
        y = y * (1.5 - 0.5 * d * y * y)
        y = y * (1.5 - 0.5 * d * y * y)
        y = y * (1.5 - 0.5 * d * y * y)
        ivm[sl] = y
        return _
    lax.fori_loop(0, SP // 16, n_body, None)

    row0 = c * NP + s * SP
    pltpu.sync_copy(ivm, inv_hbm.at[pl.ds(row0, SP)])

    def wb_body(b, _):
        pltpu.sync_copy(feats_hbm.at[pl.ds(row0 + b * RB, RB)], fvm)

        def r_body(rg, _2):
            iv16 = ivm[pl.ds(b * RB + rg * 16, 16)]
            for r in range(16):
                row = rg * 16 + r
                sv = lax.broadcast(iv16[r], (16,))
                for k in range(4):
                    sl = pl.ds(k * 16, 16)
                    wvm[row, sl] = fvm[row, sl] * sv
            return _2
        lax.fori_loop(0, RB // 16, r_body, None)
        pltpu.sync_copy(wvm, w_hbm.at[pl.ds(row0 + b * RB, RB)])
        return _
    lax.fori_loop(0, SP // RB, wb_body, None)


@functools.partial(
    pl.kernel,
    out_type=jax.ShapeDtypeStruct((2 * NP, D), jnp.float32),
    mesh=_mesh,
    scratch_types=[
        pltpu.VMEM_SHARED((ACC_ROWS, D), jnp.float32),
        pltpu.VMEM((G,), jnp.int32),
        pltpu.VMEM((G,), jnp.int32),
        pltpu.VMEM((G, D), jnp.float32),
        pltpu.VMEM((RBS, D), jnp.float32),
        pltpu.VMEM((RBS, D), jnp.float32),
        pltpu.VMEM((RBS,), jnp.float32),
        pltpu.SemaphoreType.DMA,
    ],
    compiler_params=pltpu.CompilerParams(use_tc_tiling_on_sc=False),
)
def _spmm(keys_hbm, feats_hbm, w_hbm, inv_hbm, out_hbm,
          acc, kv, ov, rows, tvm, fvm, iv, sem):
    c = lax.axis_index("c")
    s = lax.axis_index("s")
    kbase = c * EPAD + s * EPT
    obase = (1 - c) * EPAD + s * EPT
    tbl_off = (1 - c) * NP
    np_off = c * NP
    zero16 = jnp.zeros((16,), jnp.float32)

    for p in range(2):
        lo = p * HALF

        def zr(r, _):
            for k in range(4):
                tvm[r, pl.ds(k * 16, 16)] = zero16
            return _
        lax.fori_loop(0, RBS, zr, None)

        def zb(b, _):
            pltpu.sync_copy(tvm, acc.at[pl.ds(s * SD + b * RBS, RBS)])
            return _
        lax.fori_loop(0, SD // RBS, zb, None)
        plsc.subcore_barrier()

        def g_body(g, _):
            pltpu.sync_copy(keys_hbm.at[pl.ds(kbase + g * G, G)], kv)
            pltpu.sync_copy(keys_hbm.at[pl.ds(obase + g * G, G)], ov)

            def m_body(j, _2):
                sl = pl.ds(j * 16, 16)
                k = kv[sl] - lo
                inr = (k >= 0) & (k < HALF)
                kv[sl] = jnp.where(inr, k, TRASH)
                ov[sl] = ov[sl] + tbl_off
                return _2
            lax.fori_loop(0, G // 16, m_body, None)

            pltpu.async_copy(w_hbm.at[ov], rows, sem).wait()
            pltpu.sync_copy(rows, acc.at[kv], add=True)
            return _
        lax.fori_loop(0, NG, g_body, None)
        plsc.subcore_barrier()

        def d_body(b, _):
            r0 = s * SD + b * RBS
            go = np_off + lo + r0
            pltpu.sync_copy(acc.at[pl.ds(r0, RBS)], tvm)
            pltpu.sync_copy(feats_hbm.at[pl.ds(go, RBS)], fvm)
            pltpu.sync_copy(inv_hbm.at[pl.ds(go, RBS)], iv)

            def f_body(rg, _2):
                iv16 = iv[pl.ds(rg * 16, 16)] * 0.5
                for r in range(16):
                    row = rg * 16 + r
                    sv = lax.broadcast(iv16[r], (16,))
                    for k in range(4):
                        sl = pl.ds(k * 16, 16)
                        fvm[row, sl] = fvm[row, sl] * 0.5 + tvm[row, sl] * sv
                return _2
            lax.fori_loop(0, RBS // 16, f_body, None)
            pltpu.sync_copy(fvm, out_hbm.at[pl.ds(go, RBS)])
            return _
        lax.fori_loop(0, SD // RBS, d_body, None)
        plsc.subcore_barrier()


def kernel(a_feature, b_feature, edge_index):
    src = edge_index[0].astype(jnp.int32)
    dst = edge_index[1].astype(jnp.int32)
    kpad = jnp.full((EPAD - E,), PADV, jnp.int32)
    keys = jnp.concatenate([src, kpad, dst, kpad])
    fpad = jnp.zeros((NP - N, D), jnp.float32)
    feats = jnp.concatenate([a_feature, fpad, b_feature, fpad], axis=0)
    w, inv = _prep(keys, feats)
    out = _spmm(keys, feats, w, inv)
    return out[:N], out[NP:NP + N]


# speedup vs baseline: 14.0938x; 1.1250x over previous
import functools

import jax
import jax.numpy as jnp
from jax import lax
from jax.experimental import pallas as pl
from jax.experimental.pallas import tpu as pltpu
from jax.experimental.pallas import tpu_sc as plsc

N = 50000
D = 64
E = 800000

NP = 50176
SP = NP // 16
HALF = NP // 2
SD = HALF // 16
RB = 112
RBS = 32

EPAD = 819200
EPT = EPAD // 16
G = 256
NG = EPT // G
GS = 160
NGS = EPT // GS

PADV = NP - 1
TRASH = HALF
ACC_ROWS = HALF + 1

_mesh = plsc.VectorSubcoreMesh(core_axis_name="c", subcore_axis_name="s")


@functools.partial(
    pl.kernel,
    out_type=(
        jax.ShapeDtypeStruct((2 * NP, D), jnp.float32),
        jax.ShapeDtypeStruct((2 * NP,), jnp.float32),
    ),
    mesh=_mesh,
    scratch_types=[
        pltpu.VMEM_SHARED((NP,), jnp.float32),
        pltpu.VMEM((G,), jnp.int32),
        pltpu.VMEM((G,), jnp.float32),
        pltpu.VMEM((SP,), jnp.float32),
        pltpu.VMEM((SP,), jnp.float32),
        pltpu.VMEM((RB, D), jnp.float32),
        pltpu.VMEM((RB, D), jnp.float32),
        pltpu.SemaphoreType.DMA,
    ],
)
def _prep(keys_hbm, feats_hbm, w_hbm, inv_hbm,
          deg, kv, ones, dvm, ivm, fvm, wvm, sem):
    c = lax.axis_index("c")
    s = lax.axis_index("s")
    zero16 = jnp.zeros((16,), jnp.float32)

    def z_body(j, _):
        dvm[pl.ds(j * 16, 16)] = zero16
        return _
    lax.fori_loop(0, SP // 16, z_body, None)

    def o_body(j, _):
        ones[pl.ds(j * 16, 16)] = zero16 + 1.0
        return _
    lax.fori_loop(0, G // 16, o_body, None)

    pltpu.sync_copy(dvm, deg.at[pl.ds(s * SP, SP)])
    plsc.subcore_barrier()

    kbase = c * EPAD + s * EPT

    def h_body(g, _):
        pltpu.sync_copy(keys_hbm.at[pl.ds(kbase + g * G, G)], kv)
        pltpu.sync_copy(ones, deg.at[kv], add=True)
        return _
    lax.fori_loop(0, NG, h_body, None)
    plsc.subcore_barrier()

    pltpu.sync_copy(deg.at[pl.ds(s * SP, SP)], dvm)

    def n_body(j, _):
        sl = pl.ds(j * 16, 16)
        d = dvm[sl]
        di = lax.bitcast_convert_type(d, jnp.int32)
        y = lax.bitcast_convert_type(
            0x5F3759DF - lax.shift_right_logical(di, 1), jnp.float32)
        y = y * (1.5 - 0.5 * d * y * y)
        y = y * (1.5 - 0.5 * d * y * y)
        y = y * (1.5 - 0.5 * d * y * y)
        ivm[sl] = y
        return _
    lax.fori_loop(0, SP // 16, n_body, None)

    row0 = c * NP + s * SP
    pltpu.sync_copy(ivm, inv_hbm.at[pl.ds(row0, SP)])

    def wb_body(b, _):
        pltpu.sync_copy(feats_hbm.at[pl.ds(row0 + b * RB, RB)], fvm)

        def r_body(rg, _2):
            iv16 = ivm[pl.ds(b * RB + rg * 16, 16)]
            for r in range(16):
                row = rg * 16 + r
                sv = lax.broadcast(iv16[r], (16,))
                for k in range(4):
                    sl = pl.ds(k * 16, 16)
                    wvm[row, sl] = fvm[row, sl] * sv
            return _2
        lax.fori_loop(0, RB // 16, r_body, None)
        pltpu.sync_copy(wvm, w_hbm.at[pl.ds(row0 + b * RB, RB)])
        return _
    lax.fori_loop(0, SP // RB, wb_body, None)


@functools.partial(
    pl.kernel,
    out_type=jax.ShapeDtypeStruct((2 * NP, D), jnp.float32),
    mesh=_mesh,
    scratch_types=[
        pltpu.VMEM_SHARED((ACC_ROWS, D), jnp.float32),
        [pltpu.VMEM((GS,), jnp.int32)] * 4,
        [pltpu.VMEM((GS,), jnp.int32)] * 4,
        [pltpu.VMEM((GS, D), jnp.float32)] * 2,
        pltpu.VMEM((RBS, D), jnp.float32),
        pltpu.VMEM((RBS, D), jnp.float32),
        pltpu.VMEM((RBS,), jnp.float32),
        [pltpu.SemaphoreType.DMA] * 4,
        [pltpu.SemaphoreType.DMA] * 2,
    ],
    compiler_params=pltpu.CompilerParams(use_tc_tiling_on_sc=False),
)
def _spmm(keys_hbm, feats_hbm, w_hbm, inv_hbm, out_hbm,
          acc, kvs, ovs, rows, tvm, fvm, iv, semi, semr):
    c = lax.axis_index("c")
    s = lax.axis_index("s")
    kbase = c * EPAD + s * EPT
    obase = (1 - c) * EPAD + s * EPT
    tbl_off = (1 - c) * NP
    np_off = c * NP
    zero16 = jnp.zeros((16,), jnp.float32)

    def idx_descs(b, slot):
        gi = jnp.minimum(b, NGS - 1)
        return (
            pltpu.make_async_copy(
                keys_hbm.at[pl.ds(kbase + gi * GS, GS)], kvs[slot], semi[slot]),
            pltpu.make_async_copy(
                keys_hbm.at[pl.ds(obase + gi * GS, GS)], ovs[slot], semi[slot]),
        )

    def start_idx(b, slot):
        for d in idx_descs(b, slot):
            d.start()

    def wait_idx(b, slot):
        for d in idx_descs(b, slot):
            d.wait()

    for p in range(2):
        lo = p * HALF

        def zr(r, _):
            for k in range(4):
                tvm[r, pl.ds(k * 16, 16)] = zero16
            return _
        lax.fori_loop(0, RBS, zr, None)

        def zb(b, _):
            pltpu.sync_copy(tvm, acc.at[pl.ds(s * SD + b * RBS, RBS)])
            return _
        lax.fori_loop(0, SD // RBS, zb, None)
        plsc.subcore_barrier()

        def remap(slot):
            def m_body(j, _2):
                sl = pl.ds(j * 16, 16)
                k = kvs[slot][sl] - lo
                inr = (k >= 0) & (k < HALF)
                kvs[slot][sl] = jnp.where(inr, k, TRASH)
                ovs[slot][sl] = ovs[slot][sl] + tbl_off
                return _2
            lax.fori_loop(0, GS // 16, m_body, None)

        def gather(slot, rb):
            return pltpu.make_async_copy(
                w_hbm.at[ovs[slot]], rows[rb], semr[rb])

        for slot in range(4):
            start_idx(slot, slot)
        wait_idx(0, 0)
        remap(0)
        gather(0, 0).start()

        def g_body(gg, _):
            b0 = gg * 4
            for j in range(4):
                nslot = (j + 1) % 4
                wait_idx(b0 + j + 1, nslot)
                remap(nslot)
                gather(j, j % 2).wait()
                gather(nslot, (j + 1) % 2).start()
                pltpu.sync_copy(rows[j % 2], acc.at[kvs[j]], add=True)
                start_idx(b0 + j + 4, j)
            return _
        lax.fori_loop(0, NGS // 4, g_body, None)

        for slot in (1, 2, 3):
            wait_idx(NGS, slot)
        gather(0, 0).wait()
        plsc.subcore_barrier()

        def d_body(b, _):
            r0 = s * SD + b * RBS
            go = np_off + lo + r0
            pltpu.sync_copy(acc.at[pl.ds(r0, RBS)], tvm)
            pltpu.sync_copy(feats_hbm.at[pl.ds(go, RBS)], fvm)
            pltpu.sync_copy(inv_hbm.at[pl.ds(go, RBS)], iv)

            def f_body(rg, _2):
                iv16 = iv[pl.ds(rg * 16, 16)] * 0.5
                for r in range(16):
                    row = rg * 16 + r
                    sv = lax.broadcast(iv16[r], (16,))
                    for k in range(4):
                        sl = pl.ds(k * 16, 16)
                        fvm[row, sl] = fvm[row, sl] * 0.5 + tvm[row, sl] * sv
                return _2
            lax.fori_loop(0, RBS // 16, f_body, None)
            pltpu.sync_copy(fvm, out_hbm.at[pl.ds(go, RBS)])
            return _
        lax.fori_loop(0, SD // RBS, d_body, None)
        plsc.subcore_barrier()


def kernel(a_feature, b_feature, edge_index):
    src = edge_index[0].astype(jnp.int32)
    dst = edge_index[1].astype(jnp.int32)
    kpad = jnp.full((EPAD - E,), PADV, jnp.int32)
    keys = jnp.concatenate([src, kpad, dst, kpad])
    fpad = jnp.zeros((NP - N, D), jnp.float32)
    feats = jnp.concatenate([a_feature, fpad, b_feature, fpad], axis=0)
    w, inv = _prep(keys, feats)
    out = _spmm(keys, feats, w, inv)
    return out[:N], out[NP:NP + N]


# speedup vs baseline: 20.9568x; 1.4870x over previous
import functools

import jax
import jax.numpy as jnp
from jax import lax
from jax.experimental import pallas as pl
from jax.experimental.pallas import tpu as pltpu
from jax.experimental.pallas import tpu_sc as plsc

N = 50000
D = 64
E = 800000

NP = 50176
SP = NP // 16
RB = 112
NB = SP // RB

EPAD = 819200
EPT = EPAD // 16
G = 256
NG = EPT // G
GS = 256
NGS = EPT // GS

PADV = NP - 1

_mesh = plsc.VectorSubcoreMesh(core_axis_name="c", subcore_axis_name="s")


def _rne16(x):
    odd = lax.shift_right_logical(x, 16) & 1
    return lax.shift_right_logical(x + 0x7FFF + odd, 16)


@functools.partial(
    pl.kernel,
    out_type=(
        jax.ShapeDtypeStruct((2 * NP, D // 2), jnp.int32),
        jax.ShapeDtypeStruct((2 * NP,), jnp.float32),
    ),
    mesh=_mesh,
    scratch_types=[
        pltpu.VMEM_SHARED((NP,), jnp.float32),
        pltpu.VMEM((G,), jnp.int32),
        pltpu.VMEM((G,), jnp.float32),
        pltpu.VMEM((SP,), jnp.float32),
        pltpu.VMEM((SP,), jnp.float32),
        pltpu.VMEM((RB, D), jnp.float32),
        pltpu.VMEM((RB, D // 2), jnp.int32),
        pltpu.SemaphoreType.DMA,
    ],
    compiler_params=pltpu.CompilerParams(use_tc_tiling_on_sc=False),
)
def _prep(keys_hbm, feats_hbm, w_hbm, inv_hbm,
          deg, kv, ones, dvm, ivm, fvm, wvm, sem):
    c = lax.axis_index("c")
    s = lax.axis_index("s")
    zero16 = jnp.zeros((16,), jnp.float32)

    def z_body(j, _):
        dvm[pl.ds(j * 16, 16)] = zero16
        return _
    lax.fori_loop(0, SP // 16, z_body, None)

    def o_body(j, _):
        ones[pl.ds(j * 16, 16)] = zero16 + 1.0
        return _
    lax.fori_loop(0, G // 16, o_body, None)

    pltpu.sync_copy(dvm, deg.at[pl.ds(s * SP, SP)])
    plsc.subcore_barrier()

    kbase = c * EPAD + s * EPT

    def h_body(g, _):
        pltpu.sync_copy(keys_hbm.at[pl.ds(kbase + g * G, G)], kv)
        pltpu.sync_copy(ones, deg.at[kv], add=True)
        return _
    lax.fori_loop(0, NG, h_body, None)
    plsc.subcore_barrier()

    pltpu.sync_copy(deg.at[pl.ds(s * SP, SP)], dvm)

    def n_body(j, _):
        sl = pl.ds(j * 16, 16)
        d = dvm[sl]
        di = lax.bitcast_convert_type(d, jnp.int32)
        y = lax.bitcast_convert_type(
            0x5F3759DF - lax.shift_right_logical(di, 1), jnp.float32)
        y = y * (1.5 - 0.5 * d * y * y)
        y = y * (1.5 - 0.5 * d * y * y)
        y = y * (1.5 - 0.5 * d * y * y)
        ivm[sl] = y
        return _
    lax.fori_loop(0, SP // 16, n_body, None)

    row0 = c * NP + s * SP
    pltpu.sync_copy(ivm, inv_hbm.at[pl.ds(row0, SP)])

    def wb_body(b, _):
        pltpu.sync_copy(feats_hbm.at[pl.ds(row0 + b * RB, RB)], fvm)

        def r_body(rg, _2):
            iv16 = ivm[pl.ds(b * RB + rg * 16, 16)]
            for r in range(16):
                row = rg * 16 + r
                sv = lax.broadcast(iv16[r], (16,))
                for k in range(2):
                    a = fvm[row, pl.ds(k * 32, 16)] * sv
                    b2 = fvm[row, pl.ds(k * 32 + 16, 16)] * sv
                    ai = _rne16(lax.bitcast_convert_type(a, jnp.int32))
                    bi = _rne16(lax.bitcast_convert_type(b2, jnp.int32))
                    wvm[row, pl.ds(k * 16, 16)] = ai | lax.shift_left(bi, 16)
            return _2
        lax.fori_loop(0, RB // 16, r_body, None)
        pltpu.sync_copy(wvm, w_hbm.at[pl.ds(row0 + b * RB, RB)])
        return _
    lax.fori_loop(0, SP // RB, wb_body, None)


@functools.partial(
    pl.kernel,
    out_type=jax.ShapeDtypeStruct((2 * NP, D), jnp.bfloat16),
    mesh=_mesh,
    scratch_types=[
        pltpu.VMEM_SHARED((NP, D), jnp.bfloat16),
        [pltpu.VMEM((GS,), jnp.int32)] * 4,
        [pltpu.VMEM((GS,), jnp.int32)] * 4,
        [pltpu.VMEM((GS, D), jnp.bfloat16)] * 2,
        [pltpu.SemaphoreType.DMA] * 4,
        [pltpu.SemaphoreType.DMA] * 2,
    ],
    compiler_params=pltpu.CompilerParams(use_tc_tiling_on_sc=False),
)
def _spmm(keys_hbm, oths_hbm, w_hbm, t_hbm,
          acc, kvs, ovs, rows, semi, semr):
    c = lax.axis_index("c")
    s = lax.axis_index("s")
    kbase = c * EPAD + s * EPT
    zero32b = jnp.zeros((32,), jnp.bfloat16)

    def idx_descs(b, slot):
        gi = jnp.minimum(b, NGS - 1)
        return (
            pltpu.make_async_copy(
                keys_hbm.at[pl.ds(kbase + gi * GS, GS)], kvs[slot], semi[slot]),
            pltpu.make_async_copy(
                oths_hbm.at[pl.ds(kbase + gi * GS, GS)], ovs[slot], semi[slot]),
        )

    def start_idx(b, slot):
        for d in idx_descs(b, slot):
            d.start()

    def wait_idx(b, slot):
        for d in idx_descs(b, slot):
            d.wait()

    def gather(slot, rb):
        return pltpu.make_async_copy(
            w_hbm.at[ovs[slot]], rows[rb], semr[rb])

    def zr(r, _):
        for k in range(2):
            rows[0][r, pl.ds(k * 32, 32)] = zero32b
        return _
    lax.fori_loop(0, GS, zr, None)
    nz = SP // GS
    rz = (SP - nz * GS) // 32
    for b in range(nz):
        pltpu.async_copy(
            rows[0], acc.at[pl.ds(s * SP + b * GS, GS)], semr[0])
    for b in range(rz):
        pltpu.async_copy(
            rows[0].at[pl.ds(0, 32)],
            acc.at[pl.ds(s * SP + nz * GS + b * 32, 32)], semr[1])
    for b in range(nz):
        pltpu.make_async_copy(
            rows[0], acc.at[pl.ds(s * SP + b * GS, GS)], semr[0]).wait()
    for b in range(rz):
        pltpu.make_async_copy(
            rows[0].at[pl.ds(0, 32)],
            acc.at[pl.ds(s * SP + nz * GS + b * 32, 32)], semr[1]).wait()
    plsc.subcore_barrier()

    for slot in range(4):
        start_idx(slot, slot)
    wait_idx(0, 0)
    gather(0, 0).start()

    def g_body(gg, _):
        b0 = gg * 4
        for j in range(4):
            nslot = (j + 1) % 4
            wait_idx(b0 + j + 1, nslot)
            gather(j, j % 2).wait()
            gather(nslot, (j + 1) % 2).start()
            pltpu.sync_copy(rows[j % 2], acc.at[kvs[j]], add=True)
            start_idx(b0 + j + 4, j)
        return _
    lax.fori_loop(0, NGS // 4, g_body, None)

    for slot in (1, 2, 3):
        wait_idx(NGS, slot)
    gather(0, 0).wait()
    plsc.subcore_barrier()

    pltpu.sync_copy(acc.at[pl.ds(s * SP, SP)],
                    t_hbm.at[pl.ds(c * NP + s * SP, SP)])


@functools.partial(
    pl.kernel,
    out_type=jax.ShapeDtypeStruct((2 * NP, D), jnp.float32),
    mesh=_mesh,
    scratch_types=[
        [pltpu.VMEM((RB, D // 2), jnp.int32)] * 2,
        [pltpu.VMEM((RB, D), jnp.float32)] * 2,
        [pltpu.VMEM((RB,), jnp.float32)] * 2,
        [pltpu.SemaphoreType.DMA] * 2,
    ],
    compiler_params=pltpu.CompilerParams(use_tc_tiling_on_sc=False),
)
def _fin(t_hbm, feats_hbm, inv_hbm, out_hbm, tvm, fvm, iv, semd):
    c = lax.axis_index("c")
    s = lax.axis_index("s")
    base = c * NP + s * SP

    def dump_descs(b, p):
        bi = jnp.minimum(b, NB - 1)
        go = base + bi * RB
        return (
            pltpu.make_async_copy(t_hbm.at[pl.ds(go, RB)], tvm[p], semd[p]),
            pltpu.make_async_copy(feats_hbm.at[pl.ds(go, RB)], fvm[p], semd[p]),
            pltpu.make_async_copy(inv_hbm.at[pl.ds(go, RB)], iv[p], semd[p]),
        )

    def start_load(b, p):
        for dd in dump_descs(b, p):
            dd.start()

    def wait_load(b, p):
        for dd in dump_descs(b, p):
            dd.wait()

    start_load(0, 0)
    start_load(1, 1)

    def d_body(bb, _):
        for p in range(2):
            b = bb * 2 + p
            wait_load(b, p)

            def f_body(rg, _2):
                iv16 = iv[p][pl.ds(rg * 16, 16)] * 0.5
                for r in range(16):
                    row = rg * 16 + r
                    sv = lax.broadcast(iv16[r], (16,))
                    for k in range(2):
                        word = tvm[p][row, pl.ds(k * 16, 16)]
                        t0 = lax.bitcast_convert_type(
                            lax.shift_left(word, 16), jnp.float32)
                        t1 = lax.bitcast_convert_type(
                            word & jnp.int32(-65536), jnp.float32)
                        sl0 = pl.ds(k * 32, 16)
                        sl1 = pl.ds(k * 32 + 16, 16)
                        fvm[p][row, sl0] = fvm[p][row, sl0] * 0.5 + t0 * sv
                        fvm[p][row, sl1] = fvm[p][row, sl1] * 0.5 + t1 * sv
                return _2
            lax.fori_loop(0, RB // 16, f_body, None)
            go = base + b * RB
            pltpu.sync_copy(fvm[p], out_hbm.at[pl.ds(go, RB)])
            start_load(b + 2, p)
        return _
    lax.fori_loop(0, NB // 2, d_body, None)
    for p in range(2):
        wait_load(NB, p)


def kernel(a_feature, b_feature, edge_index):
    src = edge_index[0].astype(jnp.int32)
    dst = edge_index[1].astype(jnp.int32)
    kpad = jnp.full((EPAD - E,), PADV, jnp.int32)
    keys = jnp.concatenate([src, kpad, dst, kpad])
    oths = jnp.concatenate([dst + NP, kpad + NP, src, kpad])
    fpad = jnp.zeros((NP - N, D), jnp.float32)
    feats = jnp.concatenate([a_feature, fpad, b_feature, fpad], axis=0)
    w32, inv = _prep(keys, feats)
    w_bf = lax.bitcast_convert_type(w32, jnp.bfloat16).reshape(2 * NP, D)
    t_bf = _spmm(keys, oths, w_bf)
    t32 = lax.bitcast_convert_type(t_bf.reshape(2 * NP, D // 2, 2), jnp.int32)
    out = _fin(t32, feats, inv)
    return out[:N], out[NP:NP + N]


# speedup vs baseline: 22.0451x; 1.0519x over previous
import functools

import jax
import jax.numpy as jnp
from jax import lax
from jax.experimental import pallas as pl
from jax.experimental.pallas import tpu as pltpu
from jax.experimental.pallas import tpu_sc as plsc

N = 50000
D = 64
E = 800000

NP = 50176
SP = NP // 16
RB = 112
NB = SP // RB

EPT_RAW = E // 16
EPAD = 819200
EPT = EPAD // 16
GH = 200
NGH = EPT_RAW // GH
GK = 10000
NGK = EPT_RAW // GK
GS = 256
NGS = EPT // GS

PADV = NP - 1

_mesh = plsc.VectorSubcoreMesh(core_axis_name="c", subcore_axis_name="s")


def _rne16(x):
    odd = lax.shift_right_logical(x, 16) & 1
    return lax.shift_right_logical(x + 0x7FFF + odd, 16)


@functools.partial(
    pl.kernel,
    out_type=(
        jax.ShapeDtypeStruct((2 * EPAD,), jnp.int32),
        jax.ShapeDtypeStruct((2 * EPAD,), jnp.int32),
        jax.ShapeDtypeStruct((2 * NP, D // 2), jnp.int32),
        jax.ShapeDtypeStruct((2 * NP,), jnp.float32),
    ),
    mesh=_mesh,
    scratch_types=[
        pltpu.VMEM_SHARED((NP,), jnp.float32),
        pltpu.VMEM((GH,), jnp.int32),
        pltpu.VMEM((208,), jnp.float32),
        pltpu.VMEM((GK,), jnp.int32),
        pltpu.VMEM((GK,), jnp.int32),
        pltpu.VMEM((GK,), jnp.int32),
        pltpu.VMEM((SP,), jnp.float32),
        pltpu.VMEM((SP,), jnp.float32),
        pltpu.VMEM((RB, D), jnp.float32),
        pltpu.VMEM((RB, D // 2), jnp.int32),
        pltpu.SemaphoreType.DMA,
    ],
    compiler_params=pltpu.CompilerParams(use_tc_tiling_on_sc=False),
)
def _prep(edge_hbm, a_hbm, b_hbm, keys_hbm, oths_hbm, w_hbm, inv_hbm,
          deg, kv, ones, kb, ob, ov, dvm, ivm, fvm, wvm, sem):
    c = lax.axis_index("c")
    s = lax.axis_index("s")
    tbl_off = (1 - c) * NP
    ebase = s * EPT_RAW
    obase = c * EPAD + s * EPT
    zero16 = jnp.zeros((16,), jnp.float32)

    def z_body(j, _):
        dvm[pl.ds(j * 16, 16)] = zero16
        return _
    lax.fori_loop(0, SP // 16, z_body, None)

    def o_body(j, _):
        ones[pl.ds(j * 16, 16)] = zero16 + 1.0
        return _
    lax.fori_loop(0, 208 // 16, o_body, None)

    pltpu.sync_copy(dvm, deg.at[pl.ds(s * SP, SP)])

    toff16 = lax.broadcast(tbl_off, (16,))

    def k_body(g, _):
        pltpu.sync_copy(edge_hbm.at[c, pl.ds(ebase + g * GK, GK)], kb)
        pltpu.sync_copy(edge_hbm.at[1 - c, pl.ds(ebase + g * GK, GK)], ob)

        def a_body(j, _2):
            sl = pl.ds(j * 16, 16)
            ov[sl] = ob[sl] + toff16
            return _2
        lax.fori_loop(0, GK // 16, a_body, None)
        pltpu.sync_copy(kb, keys_hbm.at[pl.ds(obase + g * GK, GK)])
        pltpu.sync_copy(ov, oths_hbm.at[pl.ds(obase + g * GK, GK)])
        return _
    lax.fori_loop(0, NGK, k_body, None)

    npad = EPT - EPT_RAW
    padk16 = jnp.full((16,), PADV, jnp.int32)

    def p_body(j, _):
        sl = pl.ds(j * 16, 16)
        kb[sl] = padk16
        ov[sl] = padk16 + toff16
        return _
    lax.fori_loop(0, npad // 16, p_body, None)
    pltpu.sync_copy(kb.at[pl.ds(0, npad)],
                    keys_hbm.at[pl.ds(obase + EPT_RAW, npad)])
    pltpu.sync_copy(ov.at[pl.ds(0, npad)],
                    oths_hbm.at[pl.ds(obase + EPT_RAW, npad)])
    plsc.subcore_barrier()

    def h_body(g, _):
        pltpu.sync_copy(edge_hbm.at[c, pl.ds(ebase + g * GH, GH)], kv)
        pltpu.sync_copy(ones.at[pl.ds(0, GH)], deg.at[kv], add=True)
        return _
    lax.fori_loop(0, NGH, h_body, None)
    plsc.subcore_barrier()

    pltpu.sync_copy(deg.at[pl.ds(s * SP, SP)], dvm)

    def n_body(j, _):
        sl = pl.ds(j * 16, 16)
        d = dvm[sl]
        di = lax.bitcast_convert_type(d, jnp.int32)
        y = lax.bitcast_convert_type(
            0x5F3759DF - lax.shift_right_logical(di, 1), jnp.float32)
        y = y * (1.5 - 0.5 * d * y * y)
        y = y * (1.5 - 0.5 * d * y * y)
        y = y * (1.5 - 0.5 * d * y * y)
        ivm[sl] = y
        return _
    lax.fori_loop(0, SP // 16, n_body, None)

    pltpu.sync_copy(ivm, inv_hbm.at[pl.ds(c * NP + s * SP, SP)])

    lclamp = N - s * SP - RB

    def w_stage(feat_ref):
        def wb_body(b, _):
            loff = jnp.minimum(b * RB, lclamp)
            pltpu.sync_copy(feat_ref.at[pl.ds(s * SP + loff, RB)], fvm)

            def r_body(rg, _2):
                iv16 = ivm[pl.ds(loff + rg * 16, 16)]
                for r in range(16):
                    row = rg * 16 + r
                    sv = lax.broadcast(iv16[r], (16,))
                    for k in range(2):
                        a = fvm[row, pl.ds(k * 32, 16)] * sv
                        b2 = fvm[row, pl.ds(k * 32 + 16, 16)] * sv
                        ai = _rne16(lax.bitcast_convert_type(a, jnp.int32))
                        bi = _rne16(lax.bitcast_convert_type(b2, jnp.int32))
                        wvm[row, pl.ds(k * 16, 16)] = (
                            ai | lax.shift_left(bi, 16))
                return _2
            lax.fori_loop(0, RB // 16, r_body, None)
            pltpu.sync_copy(
                wvm, w_hbm.at[pl.ds(c * NP + s * SP + loff, RB)])
            return _
        lax.fori_loop(0, NB, wb_body, None)

    pl.when(c == 0)(lambda: w_stage(a_hbm))
    pl.when(c == 1)(lambda: w_stage(b_hbm))

    @pl.when(s == 15)
    def _zero_tail():
        zero16i = jnp.zeros((16,), jnp.int32)

        def zw_body(r, _):
            for k in range(2):
                wvm[r, pl.ds(k * 16, 16)] = zero16i
            return _
        lax.fori_loop(0, RB, zw_body, None)
        pltpu.sync_copy(wvm, w_hbm.at[pl.ds(c * NP + N, RB)])
        pltpu.sync_copy(wvm.at[pl.ds(0, NP - N - RB)],
                        w_hbm.at[pl.ds(c * NP + N + RB, NP - N - RB)])


@functools.partial(
    pl.kernel,
    out_type=jax.ShapeDtypeStruct((2 * NP, D), jnp.bfloat16),
    mesh=_mesh,
    scratch_types=[
        pltpu.VMEM_SHARED((NP, D), jnp.bfloat16),
        [pltpu.VMEM((GS,), jnp.int32)] * 4,
        [pltpu.VMEM((GS,), jnp.int32)] * 4,
        [pltpu.VMEM((GS, D), jnp.bfloat16)] * 2,
        [pltpu.SemaphoreType.DMA] * 4,
        [pltpu.SemaphoreType.DMA] * 2,
    ],
    compiler_params=pltpu.CompilerParams(use_tc_tiling_on_sc=False),
)
def _spmm(keys_hbm, oths_hbm, w_hbm, t_hbm,
          acc, kvs, ovs, rows, semi, semr):
    c = lax.axis_index("c")
    s = lax.axis_index("s")
    kbase = c * EPAD + s * EPT
    zero32b = jnp.zeros((32,), jnp.bfloat16)

    def idx_descs(b, slot):
        gi = jnp.minimum(b, NGS - 1)
        return (
            pltpu.make_async_copy(
                keys_hbm.at[pl.ds(kbase + gi * GS, GS)], kvs[slot], semi[slot]),
            pltpu.make_async_copy(
                oths_hbm.at[pl.ds(kbase + gi * GS, GS)], ovs[slot], semi[slot]),
        )

    def start_idx(b, slot):
        for d in idx_descs(b, slot):
            d.start()

    def wait_idx(b, slot):
        for d in idx_descs(b, slot):
            d.wait()

    def gather(slot, rb):
        return pltpu.make_async_copy(
            w_hbm.at[ovs[slot]], rows[rb], semr[rb])

    def zr(r, _):
        for k in range(2):
            rows[0][r, pl.ds(k * 32, 32)] = zero32b
        return _
    lax.fori_loop(0, GS, zr, None)
    nz = SP // GS
    rz = (SP - nz * GS) // 32
    for b in range(nz):
        pltpu.async_copy(
            rows[0], acc.at[pl.ds(s * SP + b * GS, GS)], semr[0])
    for b in range(rz):
        pltpu.async_copy(
            rows[0].at[pl.ds(0, 32)],
            acc.at[pl.ds(s * SP + nz * GS + b * 32, 32)], semr[1])
    for b in range(nz):
        pltpu.make_async_copy(
            rows[0], acc.at[pl.ds(s * SP + b * GS, GS)], semr[0]).wait()
    for b in range(rz):
        pltpu.make_async_copy(
            rows[0].at[pl.ds(0, 32)],
            acc.at[pl.ds(s * SP + nz * GS + b * 32, 32)], semr[1]).wait()
    plsc.subcore_barrier()

    for slot in range(4):
        start_idx(slot, slot)
    wait_idx(0, 0)
    gather(0, 0).start()

    def g_body(gg, _):
        b0 = gg * 4
        for j in range(4):
            nslot = (j + 1) % 4
            wait_idx(b0 + j + 1, nslot)
            gather(j, j % 2).wait()
            gather(nslot, (j + 1) % 2).start()
            pltpu.sync_copy(rows[j % 2], acc.at[kvs[j]], add=True)
            start_idx(b0 + j + 4, j)
        return _
    lax.fori_loop(0, NGS // 4, g_body, None)

    for slot in (1, 2, 3):
        wait_idx(NGS, slot)
    gather(0, 0).wait()
    plsc.subcore_barrier()

    pltpu.sync_copy(acc.at[pl.ds(s * SP, SP)],
                    t_hbm.at[pl.ds(c * NP + s * SP, SP)])


@functools.partial(
    pl.kernel,
    out_type=(
        jax.ShapeDtypeStruct((N, D), jnp.float32),
        jax.ShapeDtypeStruct((N, D), jnp.float32),
    ),
    mesh=_mesh,
    scratch_types=[
        [pltpu.VMEM((RB, D // 2), jnp.int32)] * 2,
        [pltpu.VMEM((RB, D), jnp.float32)] * 2,
        [pltpu.VMEM((RB,), jnp.float32)] * 2,
        [pltpu.SemaphoreType.DMA] * 2,
    ],
    compiler_params=pltpu.CompilerParams(use_tc_tiling_on_sc=False),
)
def _fin(t_hbm, a_hbm, b_hbm, inv_hbm, outa_hbm, outb_hbm,
         tvm, fvm, iv, semd):
    c = lax.axis_index("c")
    s = lax.axis_index("s")
    lclamp = N - s * SP - RB

    def fin_side(feat_ref, out_ref):
        def loffs(b):
            bi = jnp.minimum(b, NB - 1)
            return jnp.minimum(bi * RB, lclamp)

        def load_descs(b, p):
            lgo = s * SP + loffs(b)
            tgo = c * NP + lgo
            return (
                pltpu.make_async_copy(
                    t_hbm.at[pl.ds(tgo, RB)], tvm[p], semd[p]),
                pltpu.make_async_copy(
                    feat_ref.at[pl.ds(lgo, RB)], fvm[p], semd[p]),
                pltpu.make_async_copy(
                    inv_hbm.at[pl.ds(tgo, RB)], iv[p], semd[p]),
            )

        def start_load(b, p):
            for dd in load_descs(b, p):
                dd.start()

        def wait_load(b, p):
            for dd in load_descs(b, p):
                dd.wait()

        start_load(0, 0)
        start_load(1, 1)

        def d_body(bb, _):
            for p in range(2):
                b = bb * 2 + p
                wait_load(b, p)

                def f_body(rg, _2):
                    iv16 = iv[p][pl.ds(rg * 16, 16)] * 0.5
                    for r in range(16):
                        row = rg * 16 + r
                        sv = lax.broadcast(iv16[r], (16,))
                        for k in range(2):
                            word = tvm[p][row, pl.ds(k * 16, 16)]
                            t0 = lax.bitcast_convert_type(
                                lax.shift_left(word, 16), jnp.float32)
                            t1 = lax.bitcast_convert_type(
                                word & jnp.int32(-65536), jnp.float32)
                            sl0 = pl.ds(k * 32, 16)
                            sl1 = pl.ds(k * 32 + 16, 16)
                            fvm[p][row, sl0] = fvm[p][row, sl0] * 0.5 + t0 * sv
                            fvm[p][row, sl1] = fvm[p][row, sl1] * 0.5 + t1 * sv
                    return _2
                lax.fori_loop(0, RB // 16, f_body, None)
                pltpu.sync_copy(
                    fvm[p], out_ref.at[pl.ds(s * SP + loffs(b), RB)])
                start_load(b + 2, p)
            return _
        lax.fori_loop(0, NB // 2, d_body, None)
        for p in range(2):
            wait_load(NB, p)

    pl.when(c == 0)(lambda: fin_side(a_hbm, outa_hbm))
    pl.when(c == 1)(lambda: fin_side(b_hbm, outb_hbm))


def kernel(a_feature, b_feature, edge_index):
    ei = edge_index.astype(jnp.int32)
    keys, oths, w32, inv = _prep(ei, a_feature, b_feature)
    w_bf = lax.bitcast_convert_type(w32, jnp.bfloat16).reshape(2 * NP, D)
    t_bf = _spmm(keys, oths, w_bf)
    t32 = lax.bitcast_convert_type(
        t_bf.reshape(2 * NP, D // 2, 2), jnp.int32)
    return _fin(t32, a_feature, b_feature, inv)


# speedup vs baseline: 22.2199x; 1.0079x over previous
import functools

import jax
import jax.numpy as jnp
from jax import lax
from jax.experimental import pallas as pl
from jax.experimental.pallas import tpu as pltpu
from jax.experimental.pallas import tpu_sc as plsc

N = 50000
D = 64
E = 800000

NP = 50176
SP = NP // 16
RB = 112
NB = SP // RB

EPT_RAW = E // 16
EPAD = 819200
EPT = EPAD // 16
GH = 200
NGH = EPT_RAW // GH
GK = 10000
NGK = EPT_RAW // GK
GS = 320
NGS = EPT // GS

PADV = NP - 1

_mesh = plsc.VectorSubcoreMesh(core_axis_name="c", subcore_axis_name="s")


def _rne16(x):
    odd = lax.shift_right_logical(x, 16) & 1
    return lax.shift_right_logical(x + 0x7FFF + odd, 16)


@functools.partial(
    pl.kernel,
    out_type=(
        jax.ShapeDtypeStruct((2 * EPAD,), jnp.int32),
        jax.ShapeDtypeStruct((2 * EPAD,), jnp.int32),
        jax.ShapeDtypeStruct((2 * NP, D // 2), jnp.int32),
        jax.ShapeDtypeStruct((2 * NP,), jnp.float32),
    ),
    mesh=_mesh,
    scratch_types=[
        pltpu.VMEM_SHARED((NP,), jnp.float32),
        pltpu.VMEM((GH,), jnp.int32),
        pltpu.VMEM((208,), jnp.float32),
        pltpu.VMEM((GK,), jnp.int32),
        pltpu.VMEM((GK,), jnp.int32),
        pltpu.VMEM((GK,), jnp.int32),
        pltpu.VMEM((SP,), jnp.float32),
        pltpu.VMEM((SP,), jnp.float32),
        pltpu.VMEM((RB, D), jnp.float32),
        pltpu.VMEM((RB, D // 2), jnp.int32),
        pltpu.SemaphoreType.DMA,
    ],
    compiler_params=pltpu.CompilerParams(use_tc_tiling_on_sc=False),
)
def _prep(edge_hbm, a_hbm, b_hbm, keys_hbm, oths_hbm, w_hbm, inv_hbm,
          deg, kv, ones, kb, ob, ov, dvm, ivm, fvm, wvm, sem):
    c = lax.axis_index("c")
    s = lax.axis_index("s")
    tbl_off = (1 - c) * NP
    ebase = s * EPT_RAW
    obase = c * EPAD + s * EPT
    zero16 = jnp.zeros((16,), jnp.float32)

    def z_body(j, _):
        dvm[pl.ds(j * 16, 16)] = zero16
        return _
    lax.fori_loop(0, SP // 16, z_body, None)

    def o_body(j, _):
        ones[pl.ds(j * 16, 16)] = zero16 + 1.0
        return _
    lax.fori_loop(0, 208 // 16, o_body, None)

    pltpu.sync_copy(dvm, deg.at[pl.ds(s * SP, SP)])

    toff16 = lax.broadcast(tbl_off, (16,))

    def k_body(g, _):
        pltpu.sync_copy(edge_hbm.at[c, pl.ds(ebase + g * GK, GK)], kb)
        pltpu.sync_copy(edge_hbm.at[1 - c, pl.ds(ebase + g * GK, GK)], ob)

        def a_body(j, _2):
            sl = pl.ds(j * 16, 16)
            ov[sl] = ob[sl] + toff16
            return _2
        lax.fori_loop(0, GK // 16, a_body, None)
        pltpu.sync_copy(kb, keys_hbm.at[pl.ds(obase + g * GK, GK)])
        pltpu.sync_copy(ov, oths_hbm.at[pl.ds(obase + g * GK, GK)])
        return _
    lax.fori_loop(0, NGK, k_body, None)

    npad = EPT - EPT_RAW
    padk16 = jnp.full((16,), PADV, jnp.int32)

    def p_body(j, _):
        sl = pl.ds(j * 16, 16)
        kb[sl] = padk16
        ov[sl] = padk16 + toff16
        return _
    lax.fori_loop(0, npad // 16, p_body, None)
    pltpu.sync_copy(kb.at[pl.ds(0, npad)],
                    keys_hbm.at[pl.ds(obase + EPT_RAW, npad)])
    pltpu.sync_copy(ov.at[pl.ds(0, npad)],
                    oths_hbm.at[pl.ds(obase + EPT_RAW, npad)])
    plsc.subcore_barrier()

    def h_body(g, _):
        pltpu.sync_copy(edge_hbm.at[c, pl.ds(ebase + g * GH, GH)], kv)
        pltpu.sync_copy(ones.at[pl.ds(0, GH)], deg.at[kv], add=True)
        return _
    lax.fori_loop(0, NGH, h_body, None)
    plsc.subcore_barrier()

    pltpu.sync_copy(deg.at[pl.ds(s * SP, SP)], dvm)

    def n_body(j, _):
        sl = pl.ds(j * 16, 16)
        d = dvm[sl]
        di = lax.bitcast_convert_type(d, jnp.int32)
        y = lax.bitcast_convert_type(
            0x5F3759DF - lax.shift_right_logical(di, 1), jnp.float32)
        y = y * (1.5 - 0.5 * d * y * y)
        y = y * (1.5 - 0.5 * d * y * y)
        y = y * (1.5 - 0.5 * d * y * y)
        ivm[sl] = y
        return _
    lax.fori_loop(0, SP // 16, n_body, None)

    pltpu.sync_copy(ivm, inv_hbm.at[pl.ds(c * NP + s * SP, SP)])

    lclamp = N - s * SP - RB

    def w_stage(feat_ref):
        def wb_body(b, _):
            loff = jnp.minimum(b * RB, lclamp)
            pltpu.sync_copy(feat_ref.at[pl.ds(s * SP + loff, RB)], fvm)

            def r_body(rg, _2):
                iv16 = ivm[pl.ds(loff + rg * 16, 16)]
                for r in range(16):
                    row = rg * 16 + r
                    sv = lax.broadcast(iv16[r], (16,))
                    for k in range(2):
                        a = fvm[row, pl.ds(k * 32, 16)] * sv
                        b2 = fvm[row, pl.ds(k * 32 + 16, 16)] * sv
                        ai = _rne16(lax.bitcast_convert_type(a, jnp.int32))
                        bi = _rne16(lax.bitcast_convert_type(b2, jnp.int32))
                        wvm[row, pl.ds(k * 16, 16)] = (
                            ai | lax.shift_left(bi, 16))
                return _2
            lax.fori_loop(0, RB // 16, r_body, None)
            pltpu.sync_copy(
                wvm, w_hbm.at[pl.ds(c * NP + s * SP + loff, RB)])
            return _
        lax.fori_loop(0, NB, wb_body, None)

    pl.when(c == 0)(lambda: w_stage(a_hbm))
    pl.when(c == 1)(lambda: w_stage(b_hbm))

    @pl.when(s == 15)
    def _zero_tail():
        zero16i = jnp.zeros((16,), jnp.int32)

        def zw_body(r, _):
            for k in range(2):
                wvm[r, pl.ds(k * 16, 16)] = zero16i
            return _
        lax.fori_loop(0, RB, zw_body, None)
        pltpu.sync_copy(wvm, w_hbm.at[pl.ds(c * NP + N, RB)])
        pltpu.sync_copy(wvm.at[pl.ds(0, NP - N - RB)],
                        w_hbm.at[pl.ds(c * NP + N + RB, NP - N - RB)])


@functools.partial(
    pl.kernel,
    out_type=jax.ShapeDtypeStruct((2 * NP, D), jnp.bfloat16),
    mesh=_mesh,
    scratch_types=[
        pltpu.VMEM_SHARED((NP, D), jnp.bfloat16),
        [pltpu.VMEM((GS,), jnp.int32)] * 4,
        [pltpu.VMEM((GS,), jnp.int32)] * 4,
        [pltpu.VMEM((GS, D), jnp.bfloat16)] * 2,
        [pltpu.SemaphoreType.DMA] * 4,
        [pltpu.SemaphoreType.DMA] * 2,
    ],
    compiler_params=pltpu.CompilerParams(use_tc_tiling_on_sc=False),
)
def _spmm(keys_hbm, oths_hbm, w_hbm, t_hbm,
          acc, kvs, ovs, rows, semi, semr):
    c = lax.axis_index("c")
    s = lax.axis_index("s")
    kbase = c * EPAD + s * EPT
    zero32b = jnp.zeros((32,), jnp.bfloat16)

    def idx_descs(b, slot):
        gi = jnp.minimum(b, NGS - 1)
        return (
            pltpu.make_async_copy(
                keys_hbm.at[pl.ds(kbase + gi * GS, GS)], kvs[slot], semi[slot]),
            pltpu.make_async_copy(
                oths_hbm.at[pl.ds(kbase + gi * GS, GS)], ovs[slot], semi[slot]),
        )

    def start_idx(b, slot):
        for d in idx_descs(b, slot):
            d.start()

    def wait_idx(b, slot):
        for d in idx_descs(b, slot):
            d.wait()

    def gather(slot, rb):
        return pltpu.make_async_copy(
            w_hbm.at[ovs[slot]], rows[rb], semr[rb])

    def zr(r, _):
        for k in range(2):
            rows[0][r, pl.ds(k * 32, 32)] = zero32b
        return _
    lax.fori_loop(0, GS, zr, None)
    nz = SP // GS
    rz = (SP - nz * GS) // 32
    for b in range(nz):
        pltpu.async_copy(
            rows[0], acc.at[pl.ds(s * SP + b * GS, GS)], semr[0])
    for b in range(rz):
        pltpu.async_copy(
            rows[0].at[pl.ds(0, 32)],
            acc.at[pl.ds(s * SP + nz * GS + b * 32, 32)], semr[1])
    for b in range(nz):
        pltpu.make_async_copy(
            rows[0], acc.at[pl.ds(s * SP + b * GS, GS)], semr[0]).wait()
    for b in range(rz):
        pltpu.make_async_copy(
            rows[0].at[pl.ds(0, 32)],
            acc.at[pl.ds(s * SP + nz * GS + b * 32, 32)], semr[1]).wait()
    plsc.subcore_barrier()

    for slot in range(4):
        start_idx(slot, slot)
    wait_idx(0, 0)
    gather(0, 0).start()

    def g_body(gg, _):
        b0 = gg * 4
        for j in range(4):
            nslot = (j + 1) % 4
            wait_idx(b0 + j + 1, nslot)
            gather(j, j % 2).wait()
            gather(nslot, (j + 1) % 2).start()
            pltpu.sync_copy(rows[j % 2], acc.at[kvs[j]], add=True)
            start_idx(b0 + j + 4, j)
        return _
    lax.fori_loop(0, NGS // 4, g_body, None)

    for slot in (1, 2, 3):
        wait_idx(NGS, slot)
    gather(0, 0).wait()
    plsc.subcore_barrier()

    pltpu.sync_copy(acc.at[pl.ds(s * SP, SP)],
                    t_hbm.at[pl.ds(c * NP + s * SP, SP)])


@functools.partial(
    pl.kernel,
    out_type=(
        jax.ShapeDtypeStruct((N, D), jnp.float32),
        jax.ShapeDtypeStruct((N, D), jnp.float32),
    ),
    mesh=_mesh,
    scratch_types=[
        [pltpu.VMEM((RB, D // 2), jnp.int32)] * 2,
        [pltpu.VMEM((RB, D), jnp.float32)] * 2,
        [pltpu.VMEM((RB,), jnp.float32)] * 2,
        [pltpu.SemaphoreType.DMA] * 2,
    ],
    compiler_params=pltpu.CompilerParams(use_tc_tiling_on_sc=False),
)
def _fin(t_hbm, a_hbm, b_hbm, inv_hbm, outa_hbm, outb_hbm,
         tvm, fvm, iv, semd):
    c = lax.axis_index("c")
    s = lax.axis_index("s")
    lclamp = N - s * SP - RB

    def fin_side(feat_ref, out_ref):
        def loffs(b):
            bi = jnp.minimum(b, NB - 1)
            return jnp.minimum(bi * RB, lclamp)

        def load_descs(b, p):
            lgo = s * SP + loffs(b)
            tgo = c * NP + lgo
            return (
                pltpu.make_async_copy(
                    t_hbm.at[pl.ds(tgo, RB)], tvm[p], semd[p]),
                pltpu.make_async_copy(
                    feat_ref.at[pl.ds(lgo, RB)], fvm[p], semd[p]),
                pltpu.make_async_copy(
                    inv_hbm.at[pl.ds(tgo, RB)], iv[p], semd[p]),
            )

        def start_load(b, p):
            for dd in load_descs(b, p):
                dd.start()

        def wait_load(b, p):
            for dd in load_descs(b, p):
                dd.wait()

        start_load(0, 0)
        start_load(1, 1)

        def d_body(bb, _):
            for p in range(2):
                b = bb * 2 + p
                wait_load(b, p)

                def f_body(rg, _2):
                    iv16 = iv[p][pl.ds(rg * 16, 16)] * 0.5
                    for r in range(16):
                        row = rg * 16 + r
                        sv = lax.broadcast(iv16[r], (16,))
                        for k in range(2):
                            word = tvm[p][row, pl.ds(k * 16, 16)]
                            t0 = lax.bitcast_convert_type(
                                lax.shift_left(word, 16), jnp.float32)
                            t1 = lax.bitcast_convert_type(
                                word & jnp.int32(-65536), jnp.float32)
                            sl0 = pl.ds(k * 32, 16)
                            sl1 = pl.ds(k * 32 + 16, 16)
                            fvm[p][row, sl0] = fvm[p][row, sl0] * 0.5 + t0 * sv
                            fvm[p][row, sl1] = fvm[p][row, sl1] * 0.5 + t1 * sv
                    return _2
                lax.fori_loop(0, RB // 16, f_body, None)
                pltpu.sync_copy(
                    fvm[p], out_ref.at[pl.ds(s * SP + loffs(b), RB)])
                start_load(b + 2, p)
            return _
        lax.fori_loop(0, NB // 2, d_body, None)
        for p in range(2):
            wait_load(NB, p)

    pl.when(c == 0)(lambda: fin_side(a_hbm, outa_hbm))
    pl.when(c == 1)(lambda: fin_side(b_hbm, outb_hbm))


def kernel(a_feature, b_feature, edge_index):
    ei = edge_index.astype(jnp.int32)
    keys, oths, w32, inv = _prep(ei, a_feature, b_feature)
    w_bf = lax.bitcast_convert_type(w32, jnp.bfloat16).reshape(2 * NP, D)
    t_bf = _spmm(keys, oths, w_bf)
    t32 = lax.bitcast_convert_type(
        t_bf.reshape(2 * NP, D // 2, 2), jnp.int32)
    return _fin(t32, a_feature, b_feature, inv)


# speedup vs baseline: 22.2700x; 1.0023x over previous
import functools

import jax
import jax.numpy as jnp
from jax import lax
from jax.experimental import pallas as pl
from jax.experimental.pallas import tpu as pltpu
from jax.experimental.pallas import tpu_sc as plsc

N = 50000
D = 64
E = 800000

NP = 50176
SP = NP // 16
RB = 112
NB = SP // RB

EPT_RAW = E // 16
EPAD = 819200
EPT = EPAD // 16
GH = 200
NGH = EPT_RAW // GH
GK = 10000
NGK = EPT_RAW // GK
GS = 400
NGS = EPT // GS

PADV = NP - 1

_mesh = plsc.VectorSubcoreMesh(core_axis_name="c", subcore_axis_name="s")


def _rne16(x):
    odd = lax.shift_right_logical(x, 16) & 1
    return lax.shift_right_logical(x + 0x7FFF + odd, 16)


@functools.partial(
    pl.kernel,
    out_type=(
        jax.ShapeDtypeStruct((2 * EPAD,), jnp.int32),
        jax.ShapeDtypeStruct((2 * EPAD,), jnp.int32),
        jax.ShapeDtypeStruct((2 * NP, D // 2), jnp.int32),
        jax.ShapeDtypeStruct((2 * NP,), jnp.float32),
    ),
    mesh=_mesh,
    scratch_types=[
        pltpu.VMEM_SHARED((NP,), jnp.float32),
        pltpu.VMEM((GH,), jnp.int32),
        pltpu.VMEM((208,), jnp.float32),
        pltpu.VMEM((GK,), jnp.int32),
        pltpu.VMEM((GK,), jnp.int32),
        pltpu.VMEM((GK,), jnp.int32),
        pltpu.VMEM((SP,), jnp.float32),
        pltpu.VMEM((SP,), jnp.float32),
        pltpu.VMEM((RB, D), jnp.float32),
        pltpu.VMEM((RB, D // 2), jnp.int32),
        pltpu.SemaphoreType.DMA,
    ],
    compiler_params=pltpu.CompilerParams(use_tc_tiling_on_sc=False),
)
def _prep(edge_hbm, a_hbm, b_hbm, keys_hbm, oths_hbm, w_hbm, inv_hbm,
          deg, kv, ones, kb, ob, ov, dvm, ivm, fvm, wvm, sem):
    c = lax.axis_index("c")
    s = lax.axis_index("s")
    tbl_off = (1 - c) * NP
    ebase = s * EPT_RAW
    obase = c * EPAD + s * EPT
    zero16 = jnp.zeros((16,), jnp.float32)

    def z_body(j, _):
        dvm[pl.ds(j * 16, 16)] = zero16
        return _
    lax.fori_loop(0, SP // 16, z_body, None)

    def o_body(j, _):
        ones[pl.ds(j * 16, 16)] = zero16 + 1.0
        return _
    lax.fori_loop(0, 208 // 16, o_body, None)

    pltpu.sync_copy(dvm, deg.at[pl.ds(s * SP, SP)])

    toff16 = lax.broadcast(tbl_off, (16,))

    def k_body(g, _):
        pltpu.sync_copy(edge_hbm.at[c, pl.ds(ebase + g * GK, GK)], kb)
        pltpu.sync_copy(edge_hbm.at[1 - c, pl.ds(ebase + g * GK, GK)], ob)

        def a_body(j, _2):
            sl = pl.ds(j * 16, 16)
            ov[sl] = ob[sl] + toff16
            return _2
        lax.fori_loop(0, GK // 16, a_body, None)
        pltpu.sync_copy(kb, keys_hbm.at[pl.ds(obase + g * GK, GK)])
        pltpu.sync_copy(ov, oths_hbm.at[pl.ds(obase + g * GK, GK)])
        return _
    lax.fori_loop(0, NGK, k_body, None)

    npad = EPT - EPT_RAW
    padk16 = jnp.full((16,), PADV, jnp.int32)

    def p_body(j, _):
        sl = pl.ds(j * 16, 16)
        kb[sl] = padk16
        ov[sl] = padk16 + toff16
        return _
    lax.fori_loop(0, npad // 16, p_body, None)
    pltpu.sync_copy(kb.at[pl.ds(0, npad)],
                    keys_hbm.at[pl.ds(obase + EPT_RAW, npad)])
    pltpu.sync_copy(ov.at[pl.ds(0, npad)],
                    oths_hbm.at[pl.ds(obase + EPT_RAW, npad)])
    plsc.subcore_barrier()

    def h_body(g, _):
        pltpu.sync_copy(edge_hbm.at[c, pl.ds(ebase + g * GH, GH)], kv)
        pltpu.sync_copy(ones.at[pl.ds(0, GH)], deg.at[kv], add=True)
        return _
    lax.fori_loop(0, NGH, h_body, None)
    plsc.subcore_barrier()

    pltpu.sync_copy(deg.at[pl.ds(s * SP, SP)], dvm)

    def n_body(j, _):
        sl = pl.ds(j * 16, 16)
        d = dvm[sl]
        di = lax.bitcast_convert_type(d, jnp.int32)
        y = lax.bitcast_convert_type(
            0x5F3759DF - lax.shift_right_logical(di, 1), jnp.float32)
        y = y * (1.5 - 0.5 * d * y * y)
        y = y * (1.5 - 0.5 * d * y * y)
        y = y * (1.5 - 0.5 * d * y * y)
        ivm[sl] = y
        return _
    lax.fori_loop(0, SP // 16, n_body, None)

    pltpu.sync_copy(ivm, inv_hbm.at[pl.ds(c * NP + s * SP, SP)])

    lclamp = N - s * SP - RB

    def w_stage(feat_ref):
        def wb_body(b, _):
            loff = jnp.minimum(b * RB, lclamp)
            pltpu.sync_copy(feat_ref.at[pl.ds(s * SP + loff, RB)], fvm)

            def r_body(rg, _2):
                iv16 = ivm[pl.ds(loff + rg * 16, 16)]
                for r in range(16):
                    row = rg * 16 + r
                    sv = lax.broadcast(iv16[r], (16,))
                    for k in range(2):
                        a = fvm[row, pl.ds(k * 32, 16)] * sv
                        b2 = fvm[row, pl.ds(k * 32 + 16, 16)] * sv
                        ai = _rne16(lax.bitcast_convert_type(a, jnp.int32))
                        bi = _rne16(lax.bitcast_convert_type(b2, jnp.int32))
                        wvm[row, pl.ds(k * 16, 16)] = (
                            ai | lax.shift_left(bi, 16))
                return _2
            lax.fori_loop(0, RB // 16, r_body, None)
            pltpu.sync_copy(
                wvm, w_hbm.at[pl.ds(c * NP + s * SP + loff, RB)])
            return _
        lax.fori_loop(0, NB, wb_body, None)

    pl.when(c == 0)(lambda: w_stage(a_hbm))
    pl.when(c == 1)(lambda: w_stage(b_hbm))

    @pl.when(s == 15)
    def _zero_tail():
        zero16i = jnp.zeros((16,), jnp.int32)

        def zw_body(r, _):
            for k in range(2):
                wvm[r, pl.ds(k * 16, 16)] = zero16i
            return _
        lax.fori_loop(0, RB, zw_body, None)
        pltpu.sync_copy(wvm, w_hbm.at[pl.ds(c * NP + N, RB)])
        pltpu.sync_copy(wvm.at[pl.ds(0, NP - N - RB)],
                        w_hbm.at[pl.ds(c * NP + N + RB, NP - N - RB)])


@functools.partial(
    pl.kernel,
    out_type=jax.ShapeDtypeStruct((2 * NP, D), jnp.bfloat16),
    mesh=_mesh,
    scratch_types=[
        pltpu.VMEM_SHARED((NP, D), jnp.bfloat16),
        [pltpu.VMEM((GS,), jnp.int32)] * 4,
        [pltpu.VMEM((GS,), jnp.int32)] * 4,
        [pltpu.VMEM((GS, D), jnp.bfloat16)] * 2,
        [pltpu.SemaphoreType.DMA] * 4,
        [pltpu.SemaphoreType.DMA] * 2,
    ],
    compiler_params=pltpu.CompilerParams(use_tc_tiling_on_sc=False),
)
def _spmm(keys_hbm, oths_hbm, w_hbm, t_hbm,
          acc, kvs, ovs, rows, semi, semr):
    c = lax.axis_index("c")
    s = lax.axis_index("s")
    kbase = c * EPAD + s * EPT
    zero32b = jnp.zeros((32,), jnp.bfloat16)

    def idx_descs(b, slot):
        gi = jnp.minimum(b, NGS - 1)
        return (
            pltpu.make_async_copy(
                keys_hbm.at[pl.ds(kbase + gi * GS, GS)], kvs[slot], semi[slot]),
            pltpu.make_async_copy(
                oths_hbm.at[pl.ds(kbase + gi * GS, GS)], ovs[slot], semi[slot]),
        )

    def start_idx(b, slot):
        for d in idx_descs(b, slot):
            d.start()

    def wait_idx(b, slot):
        for d in idx_descs(b, slot):
            d.wait()

    def gather(slot, rb):
        return pltpu.make_async_copy(
            w_hbm.at[ovs[slot]], rows[rb], semr[rb])

    def zr(r, _):
        for k in range(2):
            rows[0][r, pl.ds(k * 32, 32)] = zero32b
        return _
    lax.fori_loop(0, GS, zr, None)
    nz = SP // GS
    rz = (SP - nz * GS) // 32
    for b in range(nz):
        pltpu.async_copy(
            rows[0], acc.at[pl.ds(s * SP + b * GS, GS)], semr[0])
    for b in range(rz):
        pltpu.async_copy(
            rows[0].at[pl.ds(0, 32)],
            acc.at[pl.ds(s * SP + nz * GS + b * 32, 32)], semr[1])
    for b in range(nz):
        pltpu.make_async_copy(
            rows[0], acc.at[pl.ds(s * SP + b * GS, GS)], semr[0]).wait()
    for b in range(rz):
        pltpu.make_async_copy(
            rows[0].at[pl.ds(0, 32)],
            acc.at[pl.ds(s * SP + nz * GS + b * 32, 32)], semr[1]).wait()
    plsc.subcore_barrier()

    for slot in range(4):
        start_idx(slot, slot)
    wait_idx(0, 0)
    gather(0, 0).start()

    def g_body(gg, _):
        b0 = gg * 4
        for j in range(4):
            nslot = (j + 1) % 4
            wait_idx(b0 + j + 1, nslot)
            gather(j, j % 2).wait()
            gather(nslot, (j + 1) % 2).start()
            pltpu.sync_copy(rows[j % 2], acc.at[kvs[j]], add=True)
            start_idx(b0 + j + 4, j)
        return _
    lax.fori_loop(0, NGS // 4, g_body, None)

    for slot in (1, 2, 3):
        wait_idx(NGS, slot)
    gather(0, 0).wait()
    plsc.subcore_barrier()

    pltpu.sync_copy(acc.at[pl.ds(s * SP, SP)],
                    t_hbm.at[pl.ds(c * NP + s * SP, SP)])


@functools.partial(
    pl.kernel,
    out_type=(
        jax.ShapeDtypeStruct((N, D), jnp.float32),
        jax.ShapeDtypeStruct((N, D), jnp.float32),
    ),
    mesh=_mesh,
    scratch_types=[
        [pltpu.VMEM((RB, D // 2), jnp.int32)] * 2,
        [pltpu.VMEM((RB, D), jnp.float32)] * 2,
        [pltpu.VMEM((RB,), jnp.float32)] * 2,
        [pltpu.SemaphoreType.DMA] * 2,
    ],
    compiler_params=pltpu.CompilerParams(use_tc_tiling_on_sc=False),
)
def _fin(t_hbm, a_hbm, b_hbm, inv_hbm, outa_hbm, outb_hbm,
         tvm, fvm, iv, semd):
    c = lax.axis_index("c")
    s = lax.axis_index("s")
    lclamp = N - s * SP - RB

    def fin_side(feat_ref, out_ref):
        def loffs(b):
            bi = jnp.minimum(b, NB - 1)
            return jnp.minimum(bi * RB, lclamp)

        def load_descs(b, p):
            lgo = s * SP + loffs(b)
            tgo = c * NP + lgo
            return (
                pltpu.make_async_copy(
                    t_hbm.at[pl.ds(tgo, RB)], tvm[p], semd[p]),
                pltpu.make_async_copy(
                    feat_ref.at[pl.ds(lgo, RB)], fvm[p], semd[p]),
                pltpu.make_async_copy(
                    inv_hbm.at[pl.ds(tgo, RB)], iv[p], semd[p]),
            )

        def start_load(b, p):
            for dd in load_descs(b, p):
                dd.start()

        def wait_load(b, p):
            for dd in load_descs(b, p):
                dd.wait()

        start_load(0, 0)
        start_load(1, 1)

        def d_body(bb, _):
            for p in range(2):
                b = bb * 2 + p
                wait_load(b, p)

                def f_body(rg, _2):
                    iv16 = iv[p][pl.ds(rg * 16, 16)] * 0.5
                    for r in range(16):
                        row = rg * 16 + r
                        sv = lax.broadcast(iv16[r], (16,))
                        for k in range(2):
                            word = tvm[p][row, pl.ds(k * 16, 16)]
                            t0 = lax.bitcast_convert_type(
                                lax.shift_left(word, 16), jnp.float32)
                            t1 = lax.bitcast_convert_type(
                                word & jnp.int32(-65536), jnp.float32)
                            sl0 = pl.ds(k * 32, 16)
                            sl1 = pl.ds(k * 32 + 16, 16)
                            fvm[p][row, sl0] = fvm[p][row, sl0] * 0.5 + t0 * sv
                            fvm[p][row, sl1] = fvm[p][row, sl1] * 0.5 + t1 * sv
                    return _2
                lax.fori_loop(0, RB // 16, f_body, None)
                pltpu.sync_copy(
                    fvm[p], out_ref.at[pl.ds(s * SP + loffs(b), RB)])
                start_load(b + 2, p)
            return _
        lax.fori_loop(0, NB // 2, d_body, None)
        for p in range(2):
            wait_load(NB, p)

    pl.when(c == 0)(lambda: fin_side(a_hbm, outa_hbm))
    pl.when(c == 1)(lambda: fin_side(b_hbm, outb_hbm))


def kernel(a_feature, b_feature, edge_index):
    ei = edge_index.astype(jnp.int32)
    keys, oths, w32, inv = _prep(ei, a_feature, b_feature)
    w_bf = lax.bitcast_convert_type(w32, jnp.bfloat16).reshape(2 * NP, D)
    t_bf = _spmm(keys, oths, w_bf)
    t32 = lax.bitcast_convert_type(
        t_bf.reshape(2 * NP, D // 2, 2), jnp.int32)
    return _fin(t32, a_feature, b_feature, inv)
